# Initial kernel scaffold; baseline (speedup 1.0000x reference)
import jax
import jax.numpy as jnp
from jax.experimental import pallas as pl


def kernel(h, pos, edge_index, batch, emb_W, emb_b, lin_W, lin_b, msg_W, msg_b, pos_W, pos_b, upd_W, upd_b, lin1_W, lin1_b, lin2_W, lin2_b):
    raise NotImplementedError("write your pallas kernel here")


# speedup vs baseline: 2.4047x; 2.4047x over previous
import functools

import jax
import jax.numpy as jnp
from jax.experimental import pallas as pl

_EMB = 256
_K = 10
_BR_KNN = 64
_NC_KNN = 10240
_BE = 640
_BR_DENSE = 512


def _dense_kernel(x_ref, w_ref, b_ref, o_ref):
    o_ref[...] = jnp.maximum(x_ref[...] @ w_ref[...] + b_ref[...], 0.0)


def _mlp_kernel(g_ref, w1_ref, b1_ref, w2_ref, b2_ref, o_ref):
    t = jnp.maximum(g_ref[...] @ w1_ref[...] + b1_ref[...], 0.0)
    o_ref[...] = t @ w2_ref[...] + b2_ref[...]


def _edge_kernel(hj_ref, dist_ref, pd_ref, lw_ref, lb_ref, mw_ref, mb_ref,
                 pw_ref, pb_ref, msg_ref, pg_ref):
    dv = jnp.maximum(dist_ref[...] * lw_ref[...] + lb_ref[...], 0.0)
    msg = jnp.maximum((hj_ref[...] + dv) @ mw_ref[...] + mb_ref[...], 0.0)
    msg_ref[...] = msg
    gate = jnp.maximum(msg @ pw_ref[...] + pb_ref[...], 0.0)
    pg_ref[...] = pd_ref[...] * gate


def _knn_kernel(rp_ref, rsq_ref, rb_ref, cx_ref, cy_ref, csq_ref, cb_ref,
                o_ref):
    i = pl.program_id(0)
    pr = rp_ref[...]
    dot = pr[:, 0:1] * cx_ref[...] + pr[:, 1:2] * cy_ref[...]
    d = (rsq_ref[...] + csq_ref[...]) - 2.0 * dot
    inf = jnp.float32(jnp.inf)
    col = jax.lax.broadcasted_iota(jnp.int32, d.shape, 1)
    row = i * _BR_KNN + jax.lax.broadcasted_iota(jnp.int32, d.shape, 0)
    d = jnp.where(rb_ref[...] != cb_ref[...], inf, d)
    d = jnp.where(row == col, inf, d)
    out = jnp.zeros((d.shape[0], 128), jnp.int32)
    ocol = jax.lax.broadcasted_iota(jnp.int32, out.shape, 1)
    for j in range(_K):
        idx = jnp.argmin(d, axis=1).astype(jnp.int32)
        out = jnp.where(ocol == j, idx[:, None], out)
        d = jnp.where(col == idx[:, None], inf, d)
    o_ref[...] = out


def _dense_relu(x, w, b):
    n = x.shape[0]
    npad = pl.cdiv(n, _BR_DENSE) * _BR_DENSE
    kin = x.shape[1]
    kpad = 128 if kin < 128 else kin
    xp = jnp.zeros((npad, kpad), jnp.float32).at[:n, :kin].set(x)
    wp = jnp.zeros((kpad, w.shape[1]), jnp.float32).at[:kin].set(w)
    out = pl.pallas_call(
        _dense_kernel,
        grid=(npad // _BR_DENSE,),
        in_specs=[
            pl.BlockSpec((_BR_DENSE, kpad), lambda i: (i, 0)),
            pl.BlockSpec((kpad, w.shape[1]), lambda i: (0, 0)),
            pl.BlockSpec((1, w.shape[1]), lambda i: (0, 0)),
        ],
        out_specs=pl.BlockSpec((_BR_DENSE, w.shape[1]), lambda i: (i, 0)),
        out_shape=jax.ShapeDtypeStruct((npad, w.shape[1]), jnp.float32),
    )(xp, wp, b[None, :])
    return out[:n]


def _edge_compute(hj, dists, pos_diff, lw, lb, mw, mb, pw, pb):
    e = hj.shape[0]
    pd_pad = jnp.zeros((e, 128), jnp.float32).at[:, :2].set(pos_diff)
    pw_pad = jnp.zeros((_EMB, 128), jnp.float32).at[:, :2].set(pw)
    pb_pad = jnp.zeros((1, 128), jnp.float32).at[0, :2].set(pb)
    msg, pg = pl.pallas_call(
        _edge_kernel,
        grid=(e // _BE,),
        in_specs=[
            pl.BlockSpec((_BE, _EMB), lambda i: (i, 0)),
            pl.BlockSpec((_BE, 1), lambda i: (i, 0)),
            pl.BlockSpec((_BE, 128), lambda i: (i, 0)),
            pl.BlockSpec((1, _EMB), lambda i: (0, 0)),
            pl.BlockSpec((1, _EMB), lambda i: (0, 0)),
            pl.BlockSpec((_EMB, _EMB), lambda i: (0, 0)),
            pl.BlockSpec((1, _EMB), lambda i: (0, 0)),
            pl.BlockSpec((_EMB, 128), lambda i: (0, 0)),
            pl.BlockSpec((1, 128), lambda i: (0, 0)),
        ],
        out_specs=[
            pl.BlockSpec((_BE, _EMB), lambda i: (i, 0)),
            pl.BlockSpec((_BE, 128), lambda i: (i, 0)),
        ],
        out_shape=[
            jax.ShapeDtypeStruct((e, _EMB), jnp.float32),
            jax.ShapeDtypeStruct((e, 128), jnp.float32),
        ],
    )(hj, dists, pd_pad, lw, lb[None, :], mw, mb[None, :], pw_pad, pb_pad)
    return msg, pg[:, :2]


def _knn_edges(pos, batch):
    n = pos.shape[0]
    nr = pl.cdiv(n, _BR_KNN) * _BR_KNN
    sq = jnp.sum(pos * pos, axis=1)
    rp = jnp.zeros((nr, 2), jnp.float32).at[:n].set(pos)
    rsq = jnp.zeros((nr, 1), jnp.float32).at[:n, 0].set(sq)
    rb = jnp.full((nr, 1), -2, jnp.int32).at[:n, 0].set(batch)
    cx = jnp.zeros((1, _NC_KNN), jnp.float32).at[0, :n].set(pos[:, 0])
    cy = jnp.zeros((1, _NC_KNN), jnp.float32).at[0, :n].set(pos[:, 1])
    csq = jnp.zeros((1, _NC_KNN), jnp.float32).at[0, :n].set(sq)
    cb = jnp.full((1, _NC_KNN), -1, jnp.int32).at[0, :n].set(batch)
    idx = pl.pallas_call(
        _knn_kernel,
        grid=(nr // _BR_KNN,),
        in_specs=[
            pl.BlockSpec((_BR_KNN, 2), lambda i: (i, 0)),
            pl.BlockSpec((_BR_KNN, 1), lambda i: (i, 0)),
            pl.BlockSpec((_BR_KNN, 1), lambda i: (i, 0)),
            pl.BlockSpec((1, _NC_KNN), lambda i: (0, 0)),
            pl.BlockSpec((1, _NC_KNN), lambda i: (0, 0)),
            pl.BlockSpec((1, _NC_KNN), lambda i: (0, 0)),
            pl.BlockSpec((1, _NC_KNN), lambda i: (0, 0)),
        ],
        out_specs=pl.BlockSpec((_BR_KNN, 128), lambda i: (i, 0)),
        out_shape=jax.ShapeDtypeStruct((nr, 128), jnp.int32),
    )(rp, rsq, rb, cx, cy, csq, cb)
    idx = idx[:n, :_K]
    src = idx.reshape(-1)
    dst = jnp.repeat(jnp.arange(n, dtype=jnp.int32), _K)
    return jnp.stack([src, dst], axis=0)


def _pair_norm(x):
    xc = x - jnp.mean(x, axis=0, keepdims=True)
    return xc / jnp.sqrt(1e-5 + jnp.mean(jnp.sum(xc * xc, axis=1)))


@jax.jit
def _run(h, pos, edge_index, batch, emb_W, emb_b, lin_W, lin_b, msg_W, msg_b,
         pos_W, pos_b, upd_W, upd_b, lin1_W, lin1_b, lin2_W, lin2_b):
    n = h.shape[0]
    hh = _dense_relu(h, emb_W, emb_b)
    ei = edge_index
    for l in range(lin_W.shape[0]):
        src = ei[0]
        dst = ei[1]
        e = src.shape[0]
        ep = pl.cdiv(e, _BE) * _BE
        src_p = jnp.zeros((ep,), jnp.int32).at[:e].set(src)
        dst_p = jnp.full((ep,), n, jnp.int32).at[:e].set(dst)
        hj = hh[src_p]
        pd = pos[dst_p.clip(0, n - 1)] - pos[src_p]
        dists = jnp.sqrt(jnp.sum(pd * pd, axis=-1) + 1e-12)[:, None]
        msg, pg = _edge_compute(hj, dists, pd, lin_W[l], lin_b[l], msg_W[l],
                                msg_b[l], pos_W[l], pos_b[l])
        msg_aggr = jnp.zeros((n, _EMB), jnp.float32).at[dst_p].add(
            msg, mode="drop")
        pos_sum = jnp.zeros((n, 2), jnp.float32).at[dst_p].add(pg, mode="drop")
        cnt = jnp.zeros((n,), jnp.float32).at[dst_p].add(
            jnp.ones((ep,), jnp.float32), mode="drop")
        pos_aggr = pos_sum / jnp.maximum(cnt, 1.0)[:, None]
        upd = _dense_relu(hh + msg_aggr, upd_W[l], upd_b[l])
        pos = pos + pos_aggr
        hh = _pair_norm(upd)
        if l == 0:
            ei = _knn_edges(pos, batch)
    g = jax.ops.segment_max(hh, batch, num_segments=64)
    g = jnp.where(jnp.isfinite(g), g, 0.0)
    w2p = jnp.zeros((_EMB, 128), jnp.float32).at[:, :2].set(lin2_W)
    b2p = jnp.zeros((1, 128), jnp.float32).at[0, :2].set(lin2_b)
    out = pl.pallas_call(
        _mlp_kernel,
        out_shape=jax.ShapeDtypeStruct((64, 128), jnp.float32),
    )(g, lin1_W, lin1_b[None, :], w2p, b2p)
    return out[:, :2]


def kernel(h, pos, edge_index, batch, emb_W, emb_b, lin_W, lin_b, msg_W,
           msg_b, pos_W, pos_b, upd_W, upd_b, lin1_W, lin1_b, lin2_W, lin2_b):
    return _run(h, pos, edge_index, batch, emb_W, emb_b, lin_W, lin_b, msg_W,
                msg_b, pos_W, pos_b, upd_W, upd_b, lin1_W, lin1_b, lin2_W,
                lin2_b)


# speedup vs baseline: 2.5907x; 1.0774x over previous
import functools

import jax
import jax.numpy as jnp
from jax.experimental import pallas as pl

_EMB = 256
_K = 10
_BR_KNN = 64
_NC_KNN = 10240
_BE = 640
_BE_KNN = 400
_BR_DENSE = 512


def _dense_kernel(x_ref, w_ref, b_ref, o_ref):
    o_ref[...] = jnp.maximum(x_ref[...] @ w_ref[...] + b_ref[...], 0.0)


def _mlp_kernel(g_ref, w1_ref, b1_ref, w2_ref, b2_ref, o_ref):
    t = jnp.maximum(g_ref[...] @ w1_ref[...] + b1_ref[...], 0.0)
    o_ref[...] = t @ w2_ref[...] + b2_ref[...]


def _edge_kernel(hj_ref, dist_ref, pd_ref, lw_ref, lb_ref, mw_ref, mb_ref,
                 pw_ref, pb_ref, msg_ref, pg_ref):
    dv = jnp.maximum(dist_ref[...] * lw_ref[...] + lb_ref[...], 0.0)
    msg = jnp.maximum((hj_ref[...] + dv) @ mw_ref[...] + mb_ref[...], 0.0)
    msg_ref[...] = msg
    gate = jnp.maximum(msg @ pw_ref[...] + pb_ref[...], 0.0)
    pg_ref[...] = pd_ref[...] * gate


def _edge_knn_kernel(hj_ref, dist_ref, pd_ref, lw_ref, lb_ref, mw_ref, mb_ref,
                     pw_ref, pb_ref, magg_ref, pagg_ref):
    dv = jnp.maximum(dist_ref[...] * lw_ref[...] + lb_ref[...], 0.0)
    msg = jnp.maximum((hj_ref[...] + dv) @ mw_ref[...] + mb_ref[...], 0.0)
    gate = jnp.maximum(msg @ pw_ref[...] + pb_ref[...], 0.0)
    pg = pd_ref[...] * gate
    rows = _BE_KNN // _K
    r = jax.lax.broadcasted_iota(jnp.int32, (rows, _BE_KNN), 0)
    c = jax.lax.broadcasted_iota(jnp.int32, (rows, _BE_KNN), 1)
    agg = jnp.where(c // _K == r, 1.0, 0.0)
    magg_ref[...] = agg @ msg
    pagg_ref[...] = agg @ pg


def _knn_kernel(rp_ref, rsq_ref, rb_ref, cx_ref, cy_ref, csq_ref, cb_ref,
                o_ref):
    i = pl.program_id(0)
    pr = rp_ref[...]
    dot = pr[:, 0:1] * cx_ref[...] + pr[:, 1:2] * cy_ref[...]
    d = (rsq_ref[...] + csq_ref[...]) - 2.0 * dot
    inf = jnp.float32(jnp.inf)
    col = jax.lax.broadcasted_iota(jnp.int32, d.shape, 1)
    row = i * _BR_KNN + jax.lax.broadcasted_iota(jnp.int32, d.shape, 0)
    d = jnp.where(rb_ref[...] != cb_ref[...], inf, d)
    d = jnp.where(row == col, inf, d)
    out = jnp.zeros((d.shape[0], 128), jnp.int32)
    ocol = jax.lax.broadcasted_iota(jnp.int32, out.shape, 1)
    for j in range(_K):
        idx = jnp.argmin(d, axis=1).astype(jnp.int32)
        out = jnp.where(ocol == j, idx[:, None], out)
        d = jnp.where(col == idx[:, None], inf, d)
    o_ref[...] = out


def _dense_relu(x, w, b):
    n = x.shape[0]
    npad = pl.cdiv(n, _BR_DENSE) * _BR_DENSE
    kin = x.shape[1]
    kpad = 128 if kin < 128 else kin
    xp = jnp.zeros((npad, kpad), jnp.float32).at[:n, :kin].set(x)
    wp = jnp.zeros((kpad, w.shape[1]), jnp.float32).at[:kin].set(w)
    out = pl.pallas_call(
        _dense_kernel,
        grid=(npad // _BR_DENSE,),
        in_specs=[
            pl.BlockSpec((_BR_DENSE, kpad), lambda i: (i, 0)),
            pl.BlockSpec((kpad, w.shape[1]), lambda i: (0, 0)),
            pl.BlockSpec((1, w.shape[1]), lambda i: (0, 0)),
        ],
        out_specs=pl.BlockSpec((_BR_DENSE, w.shape[1]), lambda i: (i, 0)),
        out_shape=jax.ShapeDtypeStruct((npad, w.shape[1]), jnp.float32),
    )(xp, wp, b[None, :])
    return out[:n]


def _edge_compute(hj, dists, pos_diff, lw, lb, mw, mb, pw, pb):
    e = hj.shape[0]
    pd_pad = jnp.zeros((e, 128), jnp.float32).at[:, :2].set(pos_diff)
    pw_pad = jnp.zeros((_EMB, 128), jnp.float32).at[:, :2].set(pw)
    pb_pad = jnp.zeros((1, 128), jnp.float32).at[0, :2].set(pb)
    msg, pg = pl.pallas_call(
        _edge_kernel,
        grid=(e // _BE,),
        in_specs=[
            pl.BlockSpec((_BE, _EMB), lambda i: (i, 0)),
            pl.BlockSpec((_BE, 1), lambda i: (i, 0)),
            pl.BlockSpec((_BE, 128), lambda i: (i, 0)),
            pl.BlockSpec((1, _EMB), lambda i: (0, 0)),
            pl.BlockSpec((1, _EMB), lambda i: (0, 0)),
            pl.BlockSpec((_EMB, _EMB), lambda i: (0, 0)),
            pl.BlockSpec((1, _EMB), lambda i: (0, 0)),
            pl.BlockSpec((_EMB, 128), lambda i: (0, 0)),
            pl.BlockSpec((1, 128), lambda i: (0, 0)),
        ],
        out_specs=[
            pl.BlockSpec((_BE, _EMB), lambda i: (i, 0)),
            pl.BlockSpec((_BE, 128), lambda i: (i, 0)),
        ],
        out_shape=[
            jax.ShapeDtypeStruct((e, _EMB), jnp.float32),
            jax.ShapeDtypeStruct((e, 128), jnp.float32),
        ],
    )(hj, dists, pd_pad, lw, lb[None, :], mw, mb[None, :], pw_pad, pb_pad)
    return msg, pg[:, :2]


def _edge_compute_knn(hj, dists, pos_diff, lw, lb, mw, mb, pw, pb):
    e = hj.shape[0]
    n_out = e // _K
    rows = _BE_KNN // _K
    pd_pad = jnp.zeros((e, 128), jnp.float32).at[:, :2].set(pos_diff)
    pw_pad = jnp.zeros((_EMB, 128), jnp.float32).at[:, :2].set(pw)
    pb_pad = jnp.zeros((1, 128), jnp.float32).at[0, :2].set(pb)
    magg, pagg = pl.pallas_call(
        _edge_knn_kernel,
        grid=(e // _BE_KNN,),
        in_specs=[
            pl.BlockSpec((_BE_KNN, _EMB), lambda i: (i, 0)),
            pl.BlockSpec((_BE_KNN, 1), lambda i: (i, 0)),
            pl.BlockSpec((_BE_KNN, 128), lambda i: (i, 0)),
            pl.BlockSpec((1, _EMB), lambda i: (0, 0)),
            pl.BlockSpec((1, _EMB), lambda i: (0, 0)),
            pl.BlockSpec((_EMB, _EMB), lambda i: (0, 0)),
            pl.BlockSpec((1, _EMB), lambda i: (0, 0)),
            pl.BlockSpec((_EMB, 128), lambda i: (0, 0)),
            pl.BlockSpec((1, 128), lambda i: (0, 0)),
        ],
        out_specs=[
            pl.BlockSpec((rows, _EMB), lambda i: (i, 0)),
            pl.BlockSpec((rows, 128), lambda i: (i, 0)),
        ],
        out_shape=[
            jax.ShapeDtypeStruct((n_out, _EMB), jnp.float32),
            jax.ShapeDtypeStruct((n_out, 128), jnp.float32),
        ],
    )(hj, dists, pd_pad, lw, lb[None, :], mw, mb[None, :], pw_pad, pb_pad)
    return magg, pagg[:, :2]


def _knn_edges(pos, batch):
    n = pos.shape[0]
    nr = pl.cdiv(n, _BR_KNN) * _BR_KNN
    sq = jnp.sum(pos * pos, axis=1)
    rp = jnp.zeros((nr, 2), jnp.float32).at[:n].set(pos)
    rsq = jnp.zeros((nr, 1), jnp.float32).at[:n, 0].set(sq)
    rb = jnp.full((nr, 1), -2, jnp.int32).at[:n, 0].set(batch)
    cx = jnp.zeros((1, _NC_KNN), jnp.float32).at[0, :n].set(pos[:, 0])
    cy = jnp.zeros((1, _NC_KNN), jnp.float32).at[0, :n].set(pos[:, 1])
    csq = jnp.zeros((1, _NC_KNN), jnp.float32).at[0, :n].set(sq)
    cb = jnp.full((1, _NC_KNN), -1, jnp.int32).at[0, :n].set(batch)
    idx = pl.pallas_call(
        _knn_kernel,
        grid=(nr // _BR_KNN,),
        in_specs=[
            pl.BlockSpec((_BR_KNN, 2), lambda i: (i, 0)),
            pl.BlockSpec((_BR_KNN, 1), lambda i: (i, 0)),
            pl.BlockSpec((_BR_KNN, 1), lambda i: (i, 0)),
            pl.BlockSpec((1, _NC_KNN), lambda i: (0, 0)),
            pl.BlockSpec((1, _NC_KNN), lambda i: (0, 0)),
            pl.BlockSpec((1, _NC_KNN), lambda i: (0, 0)),
            pl.BlockSpec((1, _NC_KNN), lambda i: (0, 0)),
        ],
        out_specs=pl.BlockSpec((_BR_KNN, 128), lambda i: (i, 0)),
        out_shape=jax.ShapeDtypeStruct((nr, 128), jnp.int32),
    )(rp, rsq, rb, cx, cy, csq, cb)
    idx = idx[:n, :_K]
    src = idx.reshape(-1)
    dst = jnp.repeat(jnp.arange(n, dtype=jnp.int32), _K)
    return jnp.stack([src, dst], axis=0)


def _pair_norm(x):
    xc = x - jnp.mean(x, axis=0, keepdims=True)
    return xc / jnp.sqrt(1e-5 + jnp.mean(jnp.sum(xc * xc, axis=1)))


@jax.jit
def _run(h, pos, edge_index, batch, emb_W, emb_b, lin_W, lin_b, msg_W, msg_b,
         pos_W, pos_b, upd_W, upd_b, lin1_W, lin1_b, lin2_W, lin2_b):
    n = h.shape[0]
    hh = _dense_relu(h, emb_W, emb_b)
    ei = edge_index
    for l in range(lin_W.shape[0]):
        src = ei[0]
        dst = ei[1]
        e = src.shape[0]
        if l == 0:
            ep = pl.cdiv(e, _BE) * _BE
            src_p = jnp.zeros((ep,), jnp.int32).at[:e].set(src)
            dst_p = jnp.full((ep,), n, jnp.int32).at[:e].set(dst)
            hj = hh[src_p]
            pd = pos[dst_p.clip(0, n - 1)] - pos[src_p]
            dists = jnp.sqrt(jnp.sum(pd * pd, axis=-1) + 1e-12)[:, None]
            msg, pg = _edge_compute(hj, dists, pd, lin_W[l], lin_b[l],
                                    msg_W[l], msg_b[l], pos_W[l], pos_b[l])
            msg_aggr = jnp.zeros((n, _EMB), jnp.float32).at[dst_p].add(
                msg, mode="drop")
            pos_sum = jnp.zeros((n, 2), jnp.float32).at[dst_p].add(
                pg, mode="drop")
            cnt = jnp.zeros((n,), jnp.float32).at[dst_p].add(
                jnp.ones((ep,), jnp.float32), mode="drop")
            pos_aggr = pos_sum / jnp.maximum(cnt, 1.0)[:, None]
        else:
            hj = hh[src]
            pd = pos[dst] - pos[src]
            dists = jnp.sqrt(jnp.sum(pd * pd, axis=-1) + 1e-12)[:, None]
            msg_aggr, pos_sum = _edge_compute_knn(
                hj, dists, pd, lin_W[l], lin_b[l], msg_W[l], msg_b[l],
                pos_W[l], pos_b[l])
            pos_aggr = pos_sum / jnp.float32(_K)
        upd = _dense_relu(hh + msg_aggr, upd_W[l], upd_b[l])
        pos = pos + pos_aggr
        hh = _pair_norm(upd)
        if l == 0:
            ei = _knn_edges(pos, batch)
    g = jax.ops.segment_max(hh, batch, num_segments=64)
    g = jnp.where(jnp.isfinite(g), g, 0.0)
    w2p = jnp.zeros((_EMB, 128), jnp.float32).at[:, :2].set(lin2_W)
    b2p = jnp.zeros((1, 128), jnp.float32).at[0, :2].set(lin2_b)
    out = pl.pallas_call(
        _mlp_kernel,
        out_shape=jax.ShapeDtypeStruct((64, 128), jnp.float32),
    )(g, lin1_W, lin1_b[None, :], w2p, b2p)
    return out[:, :2]


def kernel(h, pos, edge_index, batch, emb_W, emb_b, lin_W, lin_b, msg_W,
           msg_b, pos_W, pos_b, upd_W, upd_b, lin1_W, lin1_b, lin2_W, lin2_b):
    return _run(h, pos, edge_index, batch, emb_W, emb_b, lin_W, lin_b, msg_W,
                msg_b, pos_W, pos_b, upd_W, upd_b, lin1_W, lin1_b, lin2_W,
                lin2_b)
